# baseline (speedup 1.0000x reference)
import functools

import jax
import jax.numpy as jnp
from jax import lax
from jax.experimental import pallas as pl
from jax.experimental.pallas import tpu as pltpu
from jax.experimental.pallas import tpu_sc as plsc

BATCH = 4096
MAXLEN = 200
EMBED = 64
HIDDEN = 128
CLASS_NUM = 100

VROWS = 500000
_NC = 2
_NS = 16
_NW = _NC * _NS
_EPW = BATCH // _NW
_C0 = 128
_C1 = MAXLEN - _C0


def _pool_sc(pair_idx, half_base, table2):
    mesh = plsc.VectorSubcoreMesh(core_axis_name="c", subcore_axis_name="s")

    @functools.partial(
        pl.kernel,
        out_type=jax.ShapeDtypeStruct((BATCH, EMBED), jnp.float32),
        mesh=mesh,
        scratch_types=[
            pltpu.VMEM((_EPW * MAXLEN,), jnp.int32),
            pltpu.VMEM((_EPW * MAXLEN,), jnp.int32),
            pltpu.VMEM((MAXLEN, 128), jnp.float32),
            pltpu.VMEM((_EPW, EMBED), jnp.float32),
            pltpu.SemaphoreType.DMA,
        ],
        compiler_params=pltpu.CompilerParams(needs_layout_passes=False),
    )
    def k(idx_hbm, hb_hbm, table_hbm, out_hbm, idx_v, hb_v, buf2, out_v, sem):
        wid = lax.axis_index("s") * _NC + lax.axis_index("c")
        base = wid * _EPW
        pltpu.sync_copy(idx_hbm.at[pl.ds(base * MAXLEN, _EPW * MAXLEN)], idx_v)
        pltpu.sync_copy(hb_hbm.at[pl.ds(base * MAXLEN, _EPW * MAXLEN)], hb_v)

        scale = jnp.float32(1.0 / MAXLEN)
        lanes = lax.iota(jnp.int32, 16)

        @pl.loop(0, _EPW)
        def _elem(b):
            off = pl.multiple_of(b * MAXLEN, 8)
            d0 = pltpu.async_copy(
                table_hbm.at[idx_v.at[pl.ds(off, _C0)]],
                buf2.at[pl.ds(0, _C0)], sem)
            d1 = pltpu.async_copy(
                table_hbm.at[idx_v.at[pl.ds(off + _C0, _C1)]],
                buf2.at[pl.ds(_C0, _C1)], sem)
            d0.wait()
            d1.wait()

            zero = jnp.zeros((16,), jnp.float32)

            def red(r, accs):
                a0, a1, a2, a3 = accs
                lsplat = jnp.broadcast_to(off + r, (16,))
                h = plsc.load_gather(hb_v, [lsplat])
                rsplat = jnp.broadcast_to(r, (16,))
                i0 = h + lanes
                a0 = a0 + plsc.load_gather(buf2, [rsplat, i0])
                a1 = a1 + plsc.load_gather(buf2, [rsplat, i0 + 16])
                a2 = a2 + plsc.load_gather(buf2, [rsplat, i0 + 32])
                a3 = a3 + plsc.load_gather(buf2, [rsplat, i0 + 48])
                return (a0, a1, a2, a3)

            a0, a1, a2, a3 = lax.fori_loop(0, MAXLEN, red,
                                           (zero, zero, zero, zero))
            out_v[b, pl.ds(0, 16)] = a0 * scale
            out_v[b, pl.ds(16, 16)] = a1 * scale
            out_v[b, pl.ds(32, 16)] = a2 * scale
            out_v[b, pl.ds(48, 16)] = a3 * scale

        pltpu.sync_copy(out_v, out_hbm.at[pl.ds(base, _EPW)])

    return k(pair_idx, half_base, table2)


def _dense_body(x_ref, w1_ref, b1_ref, w2_ref, b2_ref, o_ref):
    x = x_ref[...]
    h = jnp.maximum(
        jnp.dot(x, w1_ref[...], preferred_element_type=jnp.float32)
        + b1_ref[...], 0.0)
    logits = (jnp.dot(h, w2_ref[...], preferred_element_type=jnp.float32)
              + b2_ref[...])
    m = jnp.max(logits, axis=-1, keepdims=True)
    e = jnp.exp(logits - m)
    o_ref[...] = e / jnp.sum(e, axis=-1, keepdims=True)


def _dense_tc(pooled, W1, b1, W2, b2):
    bm = 512
    grid = (BATCH // bm,)
    return pl.pallas_call(
        _dense_body,
        grid=grid,
        in_specs=[
            pl.BlockSpec((bm, EMBED), lambda i: (i, 0)),
            pl.BlockSpec((EMBED, HIDDEN), lambda i: (0, 0)),
            pl.BlockSpec((1, HIDDEN), lambda i: (0, 0)),
            pl.BlockSpec((HIDDEN, CLASS_NUM), lambda i: (0, 0)),
            pl.BlockSpec((1, CLASS_NUM), lambda i: (0, 0)),
        ],
        out_specs=pl.BlockSpec((bm, CLASS_NUM), lambda i: (i, 0)),
        out_shape=jax.ShapeDtypeStruct((BATCH, CLASS_NUM), jnp.float32),
    )(pooled, W1, b1.reshape(1, HIDDEN), W2, b2.reshape(1, CLASS_NUM))


def kernel(indices, table, W1, b1, W2, b2):
    idx_flat = indices.reshape(-1).astype(jnp.int32)
    pair_idx = idx_flat >> 1
    half_base = (idx_flat & 1) * 64
    table2 = jnp.reshape(table, (VROWS, 128))
    pooled = _pool_sc(pair_idx, half_base, table2)
    return _dense_tc(pooled, W1, b1, W2, b2)


# speedup vs baseline: 1.0683x; 1.0683x over previous
import functools

import jax
import jax.numpy as jnp
from jax import lax
from jax.experimental import pallas as pl
from jax.experimental.pallas import tpu as pltpu
from jax.experimental.pallas import tpu_sc as plsc

BATCH = 4096
MAXLEN = 200
EMBED = 64
HIDDEN = 128
CLASS_NUM = 100

VOCAB = 1000000
_TR = 512
HALFV = 977 * _TR

_NC = 2
_NS = 16
_NW = _NC * _NS
_EPW = BATCH // _NW
_C0 = 128
_C1 = MAXLEN - _C0


def _repack_body(lo_ref, hi_ref, o_ref):
    o_ref[...] = jnp.concatenate(
        [lo_ref[...].T, hi_ref[...].T], axis=1)


def _repack_tc(table_t):
    nblk = HALFV // _TR
    return pl.pallas_call(
        _repack_body,
        grid=(nblk,),
        in_specs=[
            pl.BlockSpec((EMBED, _TR), lambda i: (0, i)),
            pl.BlockSpec((EMBED, _TR), lambda i, n=nblk: (0, i + n)),
        ],
        out_specs=pl.BlockSpec((_TR, 128), lambda i: (i, 0)),
        out_shape=jax.ShapeDtypeStruct((HALFV, 128), jnp.float32),
    )(table_t, table_t)


def _pool_sc(pair_idx, half_off, table2):
    mesh = plsc.VectorSubcoreMesh(core_axis_name="c", subcore_axis_name="s")

    @functools.partial(
        pl.kernel,
        out_type=jax.ShapeDtypeStruct((BATCH, EMBED), jnp.float32),
        mesh=mesh,
        scratch_types=[
            pltpu.VMEM((_EPW * MAXLEN,), jnp.int32),
            pltpu.VMEM((_EPW * MAXLEN,), jnp.int32),
            pltpu.VMEM((MAXLEN, 128), jnp.float32),
            pltpu.VMEM((MAXLEN, 128), jnp.float32),
            pltpu.VMEM((_EPW, EMBED), jnp.float32),
            pltpu.SemaphoreType.DMA,
            pltpu.SemaphoreType.DMA,
        ],
        compiler_params=pltpu.CompilerParams(needs_layout_passes=False),
    )
    def k(idx_hbm, hb_hbm, table_hbm, out_hbm,
          idx_v, hb_v, buf0, buf1, out_v, sem0, sem1):
        wid = lax.axis_index("s") * _NC + lax.axis_index("c")
        base = wid * _EPW
        pltpu.sync_copy(idx_hbm.at[pl.ds(base * MAXLEN, _EPW * MAXLEN)], idx_v)
        pltpu.sync_copy(hb_hbm.at[pl.ds(base * MAXLEN, _EPW * MAXLEN)], hb_v)

        bufs = (buf0, buf1)
        sems = (sem0, sem1)
        scale = jnp.float32(1.0 / MAXLEN)
        lanes = lax.iota(jnp.int32, 16)

        def fire(b, s):
            off = pl.multiple_of(b * MAXLEN, 8)
            pltpu.async_copy(
                table_hbm.at[idx_v.at[pl.ds(off, _C0)]],
                bufs[s].at[pl.ds(0, _C0)], sems[s])
            pltpu.async_copy(
                table_hbm.at[idx_v.at[pl.ds(off + _C0, _C1)]],
                bufs[s].at[pl.ds(_C0, _C1)], sems[s])

        def wait(b, s):
            off = pl.multiple_of(b * MAXLEN, 8)
            pltpu.make_async_copy(
                table_hbm.at[idx_v.at[pl.ds(off, _C0)]],
                bufs[s].at[pl.ds(0, _C0)], sems[s]).wait()
            pltpu.make_async_copy(
                table_hbm.at[idx_v.at[pl.ds(off + _C0, _C1)]],
                bufs[s].at[pl.ds(_C0, _C1)], sems[s]).wait()

        fire(0, 0)
        fire(1, 1)

        @pl.loop(0, _EPW // 2)
        def _pair(p):
            for s in range(2):
                b = 2 * p + s
                wait(b, s)
                buf = bufs[s]
                off = b * MAXLEN
                zero = jnp.zeros((16,), jnp.float32)

                def red(r, accs, off=off, buf=buf):
                    a0, a1, a2, a3 = accs
                    h = plsc.load_gather(
                        hb_v, [jnp.broadcast_to(off + r, (16,))])
                    rsplat = jnp.broadcast_to(r, (16,))
                    i0 = h + lanes
                    a0 = a0 + plsc.load_gather(buf, [rsplat, i0])
                    a1 = a1 + plsc.load_gather(buf, [rsplat, i0 + 16])
                    a2 = a2 + plsc.load_gather(buf, [rsplat, i0 + 32])
                    a3 = a3 + plsc.load_gather(buf, [rsplat, i0 + 48])
                    return (a0, a1, a2, a3)

                a0, a1, a2, a3 = lax.fori_loop(0, MAXLEN, red,
                                               (zero, zero, zero, zero))

                @pl.when(b + 2 < _EPW)
                def _():
                    fire(b + 2, s)

                out_v[b, pl.ds(0, 16)] = a0 * scale
                out_v[b, pl.ds(16, 16)] = a1 * scale
                out_v[b, pl.ds(32, 16)] = a2 * scale
                out_v[b, pl.ds(48, 16)] = a3 * scale

        pltpu.sync_copy(out_v, out_hbm.at[pl.ds(base, _EPW)])

    return k(pair_idx, half_off, table2)


def _dense_body(x_ref, w1_ref, b1_ref, w2_ref, b2_ref, o_ref):
    x = x_ref[...]
    h = jnp.maximum(
        jnp.dot(x, w1_ref[...], preferred_element_type=jnp.float32)
        + b1_ref[...], 0.0)
    logits = (jnp.dot(h, w2_ref[...], preferred_element_type=jnp.float32)
              + b2_ref[...])
    m = jnp.max(logits, axis=-1, keepdims=True)
    e = jnp.exp(logits - m)
    o_ref[...] = e / jnp.sum(e, axis=-1, keepdims=True)


def _dense_tc(pooled, W1, b1, W2, b2):
    bm = 512
    grid = (BATCH // bm,)
    return pl.pallas_call(
        _dense_body,
        grid=grid,
        in_specs=[
            pl.BlockSpec((bm, EMBED), lambda i: (i, 0)),
            pl.BlockSpec((EMBED, HIDDEN), lambda i: (0, 0)),
            pl.BlockSpec((1, HIDDEN), lambda i: (0, 0)),
            pl.BlockSpec((HIDDEN, CLASS_NUM), lambda i: (0, 0)),
            pl.BlockSpec((1, CLASS_NUM), lambda i: (0, 0)),
        ],
        out_specs=pl.BlockSpec((bm, CLASS_NUM), lambda i: (i, 0)),
        out_shape=jax.ShapeDtypeStruct((BATCH, CLASS_NUM), jnp.float32),
    )(pooled, W1, b1.reshape(1, HIDDEN), W2, b2.reshape(1, CLASS_NUM))


def kernel(indices, table, W1, b1, W2, b2):
    idx_flat = indices.reshape(-1).astype(jnp.int32)
    hi = idx_flat >= HALFV
    pair_idx = jnp.where(hi, idx_flat - HALFV, idx_flat)
    half_off = jnp.where(hi, 64, 0).astype(jnp.int32)
    table2 = _repack_tc(table.T)
    pooled = _pool_sc(pair_idx, half_off, table2)
    return _dense_tc(pooled, W1, b1, W2, b2)


# speedup vs baseline: 1.6506x; 1.5450x over previous
import functools

import jax
import jax.numpy as jnp
from jax import lax
from jax.experimental import pallas as pl
from jax.experimental.pallas import tpu as pltpu
from jax.experimental.pallas import tpu_sc as plsc

BATCH = 4096
MAXLEN = 200
EMBED = 64
HIDDEN = 128
CLASS_NUM = 100

VOCAB = 1000000
_TR = 1536
HALFV = 326 * _TR

_NC = 2
_NS = 16
_NW = _NC * _NS
_EPW = BATCH // _NW
_C0 = 128
_C1 = MAXLEN - _C0


def _repack_body(lo_ref, hi_ref, o_ref):
    r = lax.broadcasted_iota(jnp.int32, (EMBED, EMBED), 0)
    c = lax.broadcasted_iota(jnp.int32, (EMBED, EMBED), 1)
    ident = (r == c).astype(jnp.float32)
    dn = (((0,), (0,)), ((), ()))
    lo_t = lax.dot_general(lo_ref[...], ident, dn,
                           preferred_element_type=jnp.float32)
    hi_t = lax.dot_general(hi_ref[...], ident, dn,
                           preferred_element_type=jnp.float32)
    o_ref[...] = jnp.concatenate([lo_t, hi_t], axis=1)


def _repack_tc(table_t):
    nblk = HALFV // _TR
    return pl.pallas_call(
        _repack_body,
        grid=(nblk,),
        in_specs=[
            pl.BlockSpec((EMBED, _TR), lambda i: (0, i)),
            pl.BlockSpec((EMBED, _TR), lambda i, n=nblk: (0, i + n)),
        ],
        out_specs=pl.BlockSpec((_TR, 128), lambda i: (i, 0)),
        out_shape=jax.ShapeDtypeStruct((HALFV, 128), jnp.float32),
    )(table_t, table_t)


def _pool_sc(pair_idx, half_off, table2):
    mesh = plsc.VectorSubcoreMesh(core_axis_name="c", subcore_axis_name="s")

    @functools.partial(
        pl.kernel,
        out_type=jax.ShapeDtypeStruct((BATCH, EMBED), jnp.float32),
        mesh=mesh,
        scratch_types=[
            pltpu.VMEM((_EPW * MAXLEN,), jnp.int32),
            pltpu.VMEM((_EPW * MAXLEN,), jnp.int32),
            pltpu.VMEM((MAXLEN, 128), jnp.float32),
            pltpu.VMEM((MAXLEN, 128), jnp.float32),
            pltpu.VMEM((_EPW, EMBED), jnp.float32),
            pltpu.SemaphoreType.DMA,
            pltpu.SemaphoreType.DMA,
        ],
        compiler_params=pltpu.CompilerParams(needs_layout_passes=False),
    )
    def k(idx_hbm, hb_hbm, table_hbm, out_hbm,
          idx_v, hb_v, buf0, buf1, out_v, sem0, sem1):
        wid = lax.axis_index("s") * _NC + lax.axis_index("c")
        base = wid * _EPW
        pltpu.sync_copy(idx_hbm.at[pl.ds(base * MAXLEN, _EPW * MAXLEN)], idx_v)
        pltpu.sync_copy(hb_hbm.at[pl.ds(base * MAXLEN, _EPW * MAXLEN)], hb_v)

        bufs = (buf0, buf1)
        sems = (sem0, sem1)
        scale = jnp.float32(1.0 / MAXLEN)
        lanes = lax.iota(jnp.int32, 16)

        def fire(b, s):
            off = pl.multiple_of(b * MAXLEN, 8)
            pltpu.async_copy(
                table_hbm.at[idx_v.at[pl.ds(off, _C0)]],
                bufs[s].at[pl.ds(0, _C0)], sems[s])
            pltpu.async_copy(
                table_hbm.at[idx_v.at[pl.ds(off + _C0, _C1)]],
                bufs[s].at[pl.ds(_C0, _C1)], sems[s])

        def wait(b, s):
            off = pl.multiple_of(b * MAXLEN, 8)
            pltpu.make_async_copy(
                table_hbm.at[idx_v.at[pl.ds(off, _C0)]],
                bufs[s].at[pl.ds(0, _C0)], sems[s]).wait()
            pltpu.make_async_copy(
                table_hbm.at[idx_v.at[pl.ds(off + _C0, _C1)]],
                bufs[s].at[pl.ds(_C0, _C1)], sems[s]).wait()

        fire(0, 0)
        fire(1, 1)

        @pl.loop(0, _EPW // 2)
        def _pair(p):
            for s in range(2):
                b = 2 * p + s
                wait(b, s)
                buf = bufs[s]
                off = b * MAXLEN
                zero = jnp.zeros((16,), jnp.float32)

                def red(r, accs, off=off, buf=buf):
                    a0, a1, a2, a3 = accs
                    h = plsc.load_gather(
                        hb_v, [jnp.broadcast_to(off + r, (16,))])
                    rsplat = jnp.broadcast_to(r, (16,))
                    i0 = h + lanes
                    a0 = a0 + plsc.load_gather(buf, [rsplat, i0])
                    a1 = a1 + plsc.load_gather(buf, [rsplat, i0 + 16])
                    a2 = a2 + plsc.load_gather(buf, [rsplat, i0 + 32])
                    a3 = a3 + plsc.load_gather(buf, [rsplat, i0 + 48])
                    return (a0, a1, a2, a3)

                a0, a1, a2, a3 = lax.fori_loop(0, MAXLEN, red,
                                               (zero, zero, zero, zero))

                @pl.when(b + 2 < _EPW)
                def _():
                    fire(b + 2, s)

                out_v[b, pl.ds(0, 16)] = a0 * scale
                out_v[b, pl.ds(16, 16)] = a1 * scale
                out_v[b, pl.ds(32, 16)] = a2 * scale
                out_v[b, pl.ds(48, 16)] = a3 * scale

        pltpu.sync_copy(out_v, out_hbm.at[pl.ds(base, _EPW)])

    return k(pair_idx, half_off, table2)


def _dense_body(x_ref, w1_ref, b1_ref, w2_ref, b2_ref, o_ref):
    x = x_ref[...]
    h = jnp.maximum(
        jnp.dot(x, w1_ref[...], preferred_element_type=jnp.float32)
        + b1_ref[...], 0.0)
    logits = (jnp.dot(h, w2_ref[...], preferred_element_type=jnp.float32)
              + b2_ref[...])
    m = jnp.max(logits, axis=-1, keepdims=True)
    e = jnp.exp(logits - m)
    o_ref[...] = e / jnp.sum(e, axis=-1, keepdims=True)


def _dense_tc(pooled, W1, b1, W2, b2):
    bm = 512
    grid = (BATCH // bm,)
    return pl.pallas_call(
        _dense_body,
        grid=grid,
        in_specs=[
            pl.BlockSpec((bm, EMBED), lambda i: (i, 0)),
            pl.BlockSpec((EMBED, HIDDEN), lambda i: (0, 0)),
            pl.BlockSpec((1, HIDDEN), lambda i: (0, 0)),
            pl.BlockSpec((HIDDEN, CLASS_NUM), lambda i: (0, 0)),
            pl.BlockSpec((1, CLASS_NUM), lambda i: (0, 0)),
        ],
        out_specs=pl.BlockSpec((bm, CLASS_NUM), lambda i: (i, 0)),
        out_shape=jax.ShapeDtypeStruct((BATCH, CLASS_NUM), jnp.float32),
    )(pooled, W1, b1.reshape(1, HIDDEN), W2, b2.reshape(1, CLASS_NUM))


def kernel(indices, table, W1, b1, W2, b2):
    idx_flat = indices.reshape(-1).astype(jnp.int32)
    hi = idx_flat >= HALFV
    pair_idx = jnp.where(hi, idx_flat - HALFV, idx_flat)
    half_off = jnp.where(hi, 64, 0).astype(jnp.int32)
    table2 = _repack_tc(table.T)
    pooled = _pool_sc(pair_idx, half_off, table2)
    return _dense_tc(pooled, W1, b1, W2, b2)


# speedup vs baseline: 1.9406x; 1.1757x over previous
import functools

import jax
import jax.numpy as jnp
from jax import lax
from jax.experimental import pallas as pl
from jax.experimental.pallas import tpu as pltpu
from jax.experimental.pallas import tpu_sc as plsc

BATCH = 4096
MAXLEN = 200
EMBED = 64
HIDDEN = 128
CLASS_NUM = 100

VOCAB = 1000000
_TR = 3072
HALFV = 163 * _TR

_NC = 2
_NS = 16
_NW = _NC * _NS
_EPW = BATCH // _NW
_C0 = 128
_C1 = MAXLEN - _C0


def _repack_body(lo_ref, hi_ref, o_ref):
    r = lax.broadcasted_iota(jnp.int32, (EMBED, EMBED), 0)
    c = lax.broadcasted_iota(jnp.int32, (EMBED, EMBED), 1)
    ident = (r == c).astype(jnp.float32)
    dn = (((0,), (0,)), ((), ()))
    lo_t = lax.dot_general(lo_ref[...], ident, dn,
                           preferred_element_type=jnp.float32)
    hi_t = lax.dot_general(hi_ref[...], ident, dn,
                           preferred_element_type=jnp.float32)
    o_ref[...] = jnp.concatenate([lo_t, hi_t], axis=1)


def _repack_tc(table_t):
    nblk = HALFV // _TR
    return pl.pallas_call(
        _repack_body,
        grid=(nblk,),
        in_specs=[
            pl.BlockSpec((EMBED, _TR), lambda i: (0, i)),
            pl.BlockSpec((EMBED, _TR), lambda i, n=nblk: (0, i + n)),
        ],
        out_specs=pl.BlockSpec((_TR, 128), lambda i: (i, 0)),
        out_shape=jax.ShapeDtypeStruct((HALFV, 128), jnp.float32),
    )(table_t, table_t)


def _pool_sc(pair_idx, half_off, table2):
    mesh = plsc.VectorSubcoreMesh(core_axis_name="c", subcore_axis_name="s")

    @functools.partial(
        pl.kernel,
        out_type=jax.ShapeDtypeStruct((BATCH, EMBED), jnp.float32),
        mesh=mesh,
        scratch_types=[
            pltpu.VMEM((_EPW * MAXLEN,), jnp.int32),
            pltpu.VMEM((_EPW * MAXLEN,), jnp.int32),
            pltpu.VMEM((MAXLEN, 128), jnp.float32),
            pltpu.VMEM((MAXLEN, 128), jnp.float32),
            pltpu.VMEM((_EPW, EMBED), jnp.float32),
            pltpu.SemaphoreType.DMA,
            pltpu.SemaphoreType.DMA,
        ],
        compiler_params=pltpu.CompilerParams(needs_layout_passes=False),
    )
    def k(idx_hbm, hb_hbm, table_hbm, out_hbm,
          idx_v, hb_v, buf0, buf1, out_v, sem0, sem1):
        wid = lax.axis_index("s") * _NC + lax.axis_index("c")
        base = wid * _EPW
        pltpu.sync_copy(idx_hbm.at[pl.ds(base * MAXLEN, _EPW * MAXLEN)], idx_v)
        pltpu.sync_copy(hb_hbm.at[pl.ds(base * MAXLEN, _EPW * MAXLEN)], hb_v)

        bufs = (buf0, buf1)
        sems = (sem0, sem1)
        scale = jnp.float32(1.0 / MAXLEN)
        lanes = lax.iota(jnp.int32, 16)

        def fire(b, s):
            off = pl.multiple_of(b * MAXLEN, 8)
            pltpu.async_copy(
                table_hbm.at[idx_v.at[pl.ds(off, _C0)]],
                bufs[s].at[pl.ds(0, _C0)], sems[s])
            pltpu.async_copy(
                table_hbm.at[idx_v.at[pl.ds(off + _C0, _C1)]],
                bufs[s].at[pl.ds(_C0, _C1)], sems[s])

        def wait(b, s):
            off = pl.multiple_of(b * MAXLEN, 8)
            pltpu.make_async_copy(
                table_hbm.at[idx_v.at[pl.ds(off, _C0)]],
                bufs[s].at[pl.ds(0, _C0)], sems[s]).wait()
            pltpu.make_async_copy(
                table_hbm.at[idx_v.at[pl.ds(off + _C0, _C1)]],
                bufs[s].at[pl.ds(_C0, _C1)], sems[s]).wait()

        fire(0, 0)
        fire(1, 1)

        @pl.loop(0, _EPW // 2)
        def _pair(p):
            for s in range(2):
                b = 2 * p + s
                wait(b, s)
                buf = bufs[s]
                off = b * MAXLEN
                zero = jnp.zeros((16,), jnp.float32)

                def red(r, accs, off=off, buf=buf):
                    a0, a1, a2, a3 = accs
                    h = plsc.load_gather(
                        hb_v, [jnp.broadcast_to(off + r, (16,))])
                    rsplat = jnp.broadcast_to(r, (16,))
                    i0 = h + lanes
                    a0 = a0 + plsc.load_gather(buf, [rsplat, i0])
                    a1 = a1 + plsc.load_gather(buf, [rsplat, i0 + 16])
                    a2 = a2 + plsc.load_gather(buf, [rsplat, i0 + 32])
                    a3 = a3 + plsc.load_gather(buf, [rsplat, i0 + 48])
                    return (a0, a1, a2, a3)

                a0, a1, a2, a3 = lax.fori_loop(0, MAXLEN, red,
                                               (zero, zero, zero, zero))

                @pl.when(b + 2 < _EPW)
                def _():
                    fire(b + 2, s)

                out_v[b, pl.ds(0, 16)] = a0 * scale
                out_v[b, pl.ds(16, 16)] = a1 * scale
                out_v[b, pl.ds(32, 16)] = a2 * scale
                out_v[b, pl.ds(48, 16)] = a3 * scale

        pltpu.sync_copy(out_v, out_hbm.at[pl.ds(base, _EPW)])

    return k(pair_idx, half_off, table2)


def _dense_body(x_ref, w1_ref, b1_ref, w2_ref, b2_ref, o_ref):
    x = x_ref[...]
    h = jnp.maximum(
        jnp.dot(x, w1_ref[...], preferred_element_type=jnp.float32)
        + b1_ref[...], 0.0)
    logits = (jnp.dot(h, w2_ref[...], preferred_element_type=jnp.float32)
              + b2_ref[...])
    m = jnp.max(logits, axis=-1, keepdims=True)
    e = jnp.exp(logits - m)
    o_ref[...] = e / jnp.sum(e, axis=-1, keepdims=True)


def _dense_tc(pooled, W1, b1, W2, b2):
    bm = 512
    grid = (BATCH // bm,)
    return pl.pallas_call(
        _dense_body,
        grid=grid,
        in_specs=[
            pl.BlockSpec((bm, EMBED), lambda i: (i, 0)),
            pl.BlockSpec((EMBED, HIDDEN), lambda i: (0, 0)),
            pl.BlockSpec((1, HIDDEN), lambda i: (0, 0)),
            pl.BlockSpec((HIDDEN, CLASS_NUM), lambda i: (0, 0)),
            pl.BlockSpec((1, CLASS_NUM), lambda i: (0, 0)),
        ],
        out_specs=pl.BlockSpec((bm, CLASS_NUM), lambda i: (i, 0)),
        out_shape=jax.ShapeDtypeStruct((BATCH, CLASS_NUM), jnp.float32),
    )(pooled, W1, b1.reshape(1, HIDDEN), W2, b2.reshape(1, CLASS_NUM))


def kernel(indices, table, W1, b1, W2, b2):
    idx_flat = indices.reshape(-1).astype(jnp.int32)
    hi = idx_flat >= HALFV
    pair_idx = jnp.where(hi, idx_flat - HALFV, idx_flat)
    half_off = jnp.where(hi, 64, 0).astype(jnp.int32)
    table2 = _repack_tc(table.T)
    pooled = _pool_sc(pair_idx, half_off, table2)
    return _dense_tc(pooled, W1, b1, W2, b2)
